# baseline (speedup 1.0000x reference)
import dataclasses
import functools

import jax
import jax.numpy as jnp
from jax import lax
from jax.experimental import pallas as pl
from jax.experimental.pallas import tpu as pltpu
from jax.experimental.pallas import tpu_sc as plsc

N = 10000
E = 320000
D_IN = 128
HID = 16
HEADS = 8
D_OUT = 16
HC = HEADS * HID

LANES = 16
EB = 128
NBLK = E // EB
NTILES = 32
ACC_ROWS = 10112
ROWS_PER_TILE = ACC_ROWS // 16

WT1_COLS = 144
ACC1_COLS = 144
WT2_COLS = 32
ACC2_COLS = 32
SCOLS = 16

_HIGH = lax.Precision.HIGHEST

_SC_PARAMS = pltpu.CompilerParams(use_tc_tiling_on_sc=False)
if "needs_layout_passes" in pltpu.CompilerParams.__dataclass_fields__:
    _SC_PARAMS = dataclasses.replace(_SC_PARAMS, needs_layout_passes=False)


def _head_select(a):
    h, f = a.shape
    rows = jnp.arange(h * f) // f
    mask = (rows[:, None] == jnp.arange(LANES)[None, :]).astype(a.dtype)
    return mask * a.reshape(-1)[:, None]


def _tc1_body(x_ref, w_ref, as_ref, at_ref, wt_ref, s_ref):
    xb = x_ref[...]
    wh = jnp.dot(xb, w_ref[...], precision=_HIGH)
    sf = jnp.dot(wh, as_ref[...], precision=_HIGH)
    tf = jnp.dot(wh, at_ref[...], precision=_HIGH)
    wt_ref[...] = jnp.concatenate([wh, tf], axis=1)
    s_ref[...] = sf


def _tc1(x, w1r, as1, at1):
    bn = 1000
    grid = (N // bn,)
    return pl.pallas_call(
        _tc1_body,
        grid=grid,
        in_specs=[
            pl.BlockSpec((bn, D_IN), lambda i: (i, 0)),
            pl.BlockSpec((D_IN, HC), lambda i: (0, 0)),
            pl.BlockSpec((HC, LANES), lambda i: (0, 0)),
            pl.BlockSpec((HC, LANES), lambda i: (0, 0)),
        ],
        out_specs=[
            pl.BlockSpec((bn, WT1_COLS), lambda i: (i, 0)),
            pl.BlockSpec((bn, SCOLS), lambda i: (i, 0)),
        ],
        out_shape=[
            jax.ShapeDtypeStruct((N, WT1_COLS), jnp.float32),
            jax.ShapeDtypeStruct((N, SCOLS), jnp.float32),
        ],
    )(x, w1r, as1, at1)


def _tc2_body(p_ref, b1_ref, w2_ref, as_ref, at_ref, wt_ref, s_ref):
    acc = p_ref[0] + p_ref[1]
    numer = acc[:, 0:HC]
    den = acc[:, HC:WT1_COLS]
    dsafe = jnp.where(den == 0.0, 1.0, den)
    rid = lax.broadcasted_iota(jnp.int32, (LANES, HC), 0)
    cid = lax.broadcasted_iota(jnp.int32, (LANES, HC), 1)
    rep = jnp.where((cid // HID) == rid, 1.0, 0.0)
    denb = jnp.dot(dsafe, rep, precision=_HIGH)
    v = numer / denb + b1_ref[...]
    h1 = jnp.where(v > 0.0, v, jnp.exp(v) - 1.0)
    wh2 = jnp.dot(h1, w2_ref[...], precision=_HIGH)
    sf = jnp.dot(wh2, as_ref[...], precision=_HIGH)
    tf = jnp.dot(wh2, at_ref[...], precision=_HIGH)
    wt_ref[...] = jnp.concatenate([wh2, tf], axis=1)
    s_ref[...] = sf


def _tc2(part1, b1, w2r, as2, at2):
    bn = 1000
    grid = (N // bn,)
    return pl.pallas_call(
        _tc2_body,
        grid=grid,
        in_specs=[
            pl.BlockSpec((2, bn, ACC1_COLS), lambda i: (0, i, 0)),
            pl.BlockSpec((1, HC), lambda i: (0, 0)),
            pl.BlockSpec((HC, D_OUT), lambda i: (0, 0)),
            pl.BlockSpec((D_OUT, LANES), lambda i: (0, 0)),
            pl.BlockSpec((D_OUT, LANES), lambda i: (0, 0)),
        ],
        out_specs=[
            pl.BlockSpec((bn, WT2_COLS), lambda i: (i, 0)),
            pl.BlockSpec((bn, SCOLS), lambda i: (i, 0)),
        ],
        out_shape=[
            jax.ShapeDtypeStruct((N, WT2_COLS), jnp.float32),
            jax.ShapeDtypeStruct((N, SCOLS), jnp.float32),
        ],
    )(part1, b1, w2r, as2, at2)


def _zeros16():
    return jnp.zeros((LANES,), jnp.float32)


EPT = E // NTILES


def _sc_edge(ei, wt, s, wt_cols, acc_cols, heads, t_col, ebp):
    nj = EPT // ebp
    assert EPT % ebp == 0 and ebp % 8 == 0
    mesh = plsc.VectorSubcoreMesh(core_axis_name="c", subcore_axis_name="s")

    @functools.partial(
        pl.kernel,
        out_type=jax.ShapeDtypeStruct((2, ACC_ROWS, acc_cols), jnp.float32),
        mesh=mesh,
        scratch_types=[
            pltpu.VMEM((2, ebp), jnp.int32),
            pltpu.VMEM((2, ebp), jnp.int32),
            pltpu.VMEM((2, ebp), jnp.int32),
            pltpu.VMEM((2, ebp, SCOLS), jnp.float32),
            pltpu.VMEM((2, ebp, wt_cols), jnp.float32),
            pltpu.VMEM((2, ebp, acc_cols), jnp.float32),
            pltpu.VMEM_SHARED((ACC_ROWS, acc_cols), jnp.float32),
            pltpu.SemaphoreType.DMA,
            pltpu.SemaphoreType.DMA,
            pltpu.SemaphoreType.DMA,
            pltpu.SemaphoreType.DMA,
            pltpu.SemaphoreType.DMA,
            pltpu.SemaphoreType.DMA,
        ],
        compiler_params=_SC_PARAMS,
    )
    def edge_kernel(ei_hbm, wt_hbm, s_hbm, out_hbm,
                    dstb_v, srcb_v, dsts_v, sb_v, wt_v, msg_v, acc_sh,
                    isem0, isem1, gsem0, gsem1, ssem0, ssem1):
        cid = lax.axis_index("c")
        sid = lax.axis_index("s")
        wid = cid * 16 + sid
        isems = (isem0, isem1)
        gsems = (gsem0, gsem1)
        ssems = (ssem0, ssem1)
        ebase = wid * EPT

        zb = msg_v.at[0]

        @pl.loop(0, ebp)
        def _(r):
            for c in range(0, acc_cols, LANES):
                zb[r, pl.ds(c, LANES)] = _zeros16()

        base_row = sid * ROWS_PER_TILE
        full, rem = divmod(ROWS_PER_TILE, ebp)
        for i in range(full):
            pltpu.sync_copy(zb, acc_sh.at[pl.ds(base_row + i * ebp, ebp)])
        if rem:
            pltpu.sync_copy(zb.at[pl.ds(0, rem)],
                            acc_sh.at[pl.ds(base_row + full * ebp, rem)])
        plsc.subcore_barrier()

        def start_idx(slot, jb):
            base = pl.multiple_of(ebase + jb * ebp, 8)
            pltpu.async_copy(ei_hbm.at[0, pl.ds(base, ebp)],
                             dstb_v.at[slot], isems[slot])
            pltpu.async_copy(ei_hbm.at[1, pl.ds(base, ebp)],
                             srcb_v.at[slot], isems[slot])

        def wait_idx(slot, jb):
            base = pl.multiple_of(ebase + jb * ebp, 8)
            pltpu.make_async_copy(ei_hbm.at[0, pl.ds(base, ebp)],
                                  dstb_v.at[slot], isems[slot]).wait()
            pltpu.make_async_copy(ei_hbm.at[1, pl.ds(base, ebp)],
                                  srcb_v.at[slot], isems[slot]).wait()

        def fetch(slot):
            pltpu.async_copy(s_hbm.at[dstb_v.at[slot]], sb_v.at[slot],
                             gsems[slot])
            pltpu.async_copy(wt_hbm.at[srcb_v.at[slot]], wt_v.at[slot],
                             gsems[slot])

        def wait_fetch(slot):
            pltpu.make_async_copy(s_hbm.at[dstb_v.at[slot]], sb_v.at[slot],
                                  gsems[slot]).wait()
            pltpu.make_async_copy(wt_hbm.at[srcb_v.at[slot]], wt_v.at[slot],
                                  gsems[slot]).wait()

        _copy_offs = sorted(set(list(range(0, ebp - 15, 16)) + [ebp - 16]))

        def snap_idx(slot):
            for off in _copy_offs:
                dsts_v.at[slot][pl.ds(off, LANES)] = \
                    dstb_v.at[slot][pl.ds(off, LANES)]

        def compute(slot):
            sbs, wts, msgs = sb_v.at[slot], wt_v.at[slot], msg_v.at[slot]

            @plsc.parallel_loop(0, ebp, unroll=8)
            def _(k):
                sv = sbs[k, pl.ds(0, LANES)]
                tv = wts[k, pl.ds(t_col, LANES)]
                z = sv + tv
                w = jnp.exp(jnp.maximum(z, 0.2 * z))
                msgs[k, pl.ds(t_col, LANES)] = w
                for h in range(heads):
                    wspl = lax.gather(
                        w, jnp.full((LANES, 1), h, jnp.int32),
                        lax.GatherDimensionNumbers(
                            offset_dims=(), collapsed_slice_dims=(0,),
                            start_index_map=(0,)),
                        slice_sizes=(1,),
                        mode=lax.GatherScatterMode.PROMISE_IN_BOUNDS)
                    sl = pl.ds(h * LANES, LANES)
                    msgs[k, sl] = wts[k, sl] * wspl

        def start_scatter(slot):
            pltpu.async_copy(msg_v.at[slot], acc_sh.at[dsts_v.at[slot]],
                             ssems[slot], add=True)

        def wait_scatter(slot):
            pltpu.make_async_copy(msg_v.at[slot],
                                  acc_sh.at[dsts_v.at[slot]],
                                  ssems[slot]).wait()

        start_idx(0, jnp.int32(0))
        start_idx(1, jnp.int32(1))
        wait_idx(0, jnp.int32(0))
        fetch(0)
        wait_idx(1, jnp.int32(1))
        fetch(1)

        @pl.loop(0, (nj - 1) // 2)
        def _(t):
            j0 = 2 * t
            j1 = 2 * t + 1
            wait_fetch(0)

            @pl.when(t > 0)
            def _():
                wait_scatter(0)

            snap_idx(0)
            start_idx(0, j0 + 2)
            compute(0)
            start_scatter(0)
            wait_idx(0, j0 + 2)
            fetch(0)

            wait_fetch(1)

            @pl.when(t > 0)
            def _():
                wait_scatter(1)

            snap_idx(1)

            @pl.when(j1 + 2 < nj)
            def _():
                start_idx(1, j1 + 2)

            compute(1)
            start_scatter(1)

            @pl.when(j1 + 2 < nj)
            def _():
                wait_idx(1, j1 + 2)
                fetch(1)

        if nj % 2:
            wait_fetch(0)
            wait_scatter(0)
            snap_idx(0)
            compute(0)
            start_scatter(0)
            wait_scatter(1)
            wait_scatter(0)
        else:
            wait_fetch(0)
            wait_scatter(0)
            snap_idx(0)
            compute(0)
            start_scatter(0)
            wait_fetch(1)
            wait_scatter(1)
            snap_idx(1)
            compute(1)
            start_scatter(1)
            wait_scatter(0)
            wait_scatter(1)
        plsc.subcore_barrier()
        pltpu.sync_copy(acc_sh.at[pl.ds(base_row, ROWS_PER_TILE)],
                        out_hbm.at[cid, pl.ds(base_row, ROWS_PER_TILE)])

    return edge_kernel(ei, wt, s)


def _sc_final(p2a, p2b, index, b2):
    mesh = plsc.VectorSubcoreMesh(core_axis_name="c", subcore_axis_name="s")
    KB = 80
    nblk = N // KB

    @functools.partial(
        pl.kernel,
        out_type=jax.ShapeDtypeStruct((N, D_OUT), jnp.float32),
        mesh=mesh,
        scratch_types=[
            pltpu.VMEM((KB,), jnp.int32),
            pltpu.VMEM((KB, ACC2_COLS), jnp.float32),
            pltpu.VMEM((KB, ACC2_COLS), jnp.float32),
            pltpu.VMEM((KB, D_OUT), jnp.float32),
            pltpu.VMEM((KB, LANES), jnp.float32),
            pltpu.VMEM((LANES,), jnp.float32),
        ],
        compiler_params=_SC_PARAMS,
    )
    def final_kernel(pa_hbm, pb_hbm, idx_hbm, b2_hbm, out_hbm,
                     i_v, ra_v, rb_v, o_v, d_v, b2_v):
        cid = lax.axis_index("c")
        sid = lax.axis_index("s")
        wid = cid * 16 + sid
        pltpu.sync_copy(b2_hbm, b2_v)

        nrounds = (nblk + NTILES - 1) // NTILES

        @pl.loop(0, nrounds)
        def _(j):
            b = j * NTILES + wid

            @pl.when(b < nblk)
            def _():
                base = b * KB
                pltpu.sync_copy(idx_hbm.at[pl.ds(base, KB)], i_v)
                pltpu.sync_copy(pa_hbm.at[i_v], ra_v)
                pltpu.sync_copy(pb_hbm.at[i_v], rb_v)

                @pl.loop(0, KB)
                def _(k):
                    nv = ra_v[k, pl.ds(0, LANES)] + rb_v[k, pl.ds(0, LANES)]
                    dv = (ra_v[k, pl.ds(D_OUT, LANES)]
                          + rb_v[k, pl.ds(D_OUT, LANES)])
                    d_v[k, pl.ds(0, LANES)] = dv
                    i0 = jnp.full((LANES,), k, jnp.int32)
                    i1 = jnp.zeros((LANES,), jnp.int32)
                    dspl = plsc.load_gather(d_v, [i0, i1])
                    dsafe = jnp.where(dspl == 0.0, 1.0, dspl)
                    o_v[k, pl.ds(0, LANES)] = nv / dsafe + b2_v[pl.ds(0, LANES)]

                pltpu.sync_copy(o_v, out_hbm.at[pl.ds(base, KB)])

    return final_kernel(p2a, p2b, index, b2)


def kernel(x, edge_index, index, W1, a_s1, a_n1, b1, W2, a_s2, a_n2, b2):
    ei = edge_index.astype(jnp.int32)
    idx = index.astype(jnp.int32)
    w1r = jnp.transpose(W1, (1, 0, 2)).reshape(D_IN, HC)
    as1 = _head_select(a_s1)
    at1 = _head_select(a_n1)
    w2r = W2.reshape(HC, D_OUT)
    as2 = _head_select(a_s2)
    at2 = _head_select(a_n2)

    wt1, s1 = _tc1(x, w1r, as1, at1)
    part1 = _sc_edge(ei, wt1, s1, WT1_COLS, ACC1_COLS, HEADS, HC, 40)
    wt2, s2 = _tc2(part1[:, :N, :], b1.reshape(1, HC), w2r, as2, at2)
    part2 = _sc_edge(ei, wt2, s2, WT2_COLS, ACC2_COLS, 1, D_OUT, 80)
    return _sc_final(part2[0], part2[1], idx, b2)


# speedup vs baseline: 1.0000x; 1.0000x over previous
import dataclasses
import functools

import jax
import jax.numpy as jnp
from jax import lax
from jax.experimental import pallas as pl
from jax.experimental.pallas import tpu as pltpu
from jax.experimental.pallas import tpu_sc as plsc

N = 10000
E = 320000
D_IN = 128
HID = 16
HEADS = 8
D_OUT = 16
HC = HEADS * HID

LANES = 16
EB = 128
NBLK = E // EB
NTILES = 32
ACC_ROWS = 10112
ROWS_PER_TILE = ACC_ROWS // 16

WT1_COLS = 144
ACC1_COLS = 144
WT2_COLS = 32
ACC2_COLS = 32
SCOLS = 16

_HIGH = lax.Precision.HIGHEST

_SC_PARAMS = pltpu.CompilerParams(use_tc_tiling_on_sc=False)
if "needs_layout_passes" in pltpu.CompilerParams.__dataclass_fields__:
    _SC_PARAMS = dataclasses.replace(_SC_PARAMS, needs_layout_passes=False)


def _head_select(a):
    h, f = a.shape
    rows = jnp.arange(h * f) // f
    mask = (rows[:, None] == jnp.arange(LANES)[None, :]).astype(a.dtype)
    return mask * a.reshape(-1)[:, None]


def _tc1_body(x_ref, w_ref, as_ref, at_ref, wt_ref, s_ref):
    xb = x_ref[...]
    wh = jnp.dot(xb, w_ref[...], precision=_HIGH)
    sf = jnp.dot(wh, as_ref[...], precision=_HIGH)
    tf = jnp.dot(wh, at_ref[...], precision=_HIGH)
    wt_ref[...] = jnp.concatenate([wh, tf], axis=1)
    s_ref[...] = sf


def _tc1(x, w1r, as1, at1):
    bn = 1000
    grid = (N // bn,)
    return pl.pallas_call(
        _tc1_body,
        grid=grid,
        in_specs=[
            pl.BlockSpec((bn, D_IN), lambda i: (i, 0)),
            pl.BlockSpec((D_IN, HC), lambda i: (0, 0)),
            pl.BlockSpec((HC, LANES), lambda i: (0, 0)),
            pl.BlockSpec((HC, LANES), lambda i: (0, 0)),
        ],
        out_specs=[
            pl.BlockSpec((bn, WT1_COLS), lambda i: (i, 0)),
            pl.BlockSpec((bn, SCOLS), lambda i: (i, 0)),
        ],
        out_shape=[
            jax.ShapeDtypeStruct((N, WT1_COLS), jnp.float32),
            jax.ShapeDtypeStruct((N, SCOLS), jnp.float32),
        ],
    )(x, w1r, as1, at1)


def _tc2_body(p_ref, b1_ref, w2_ref, as_ref, at_ref, wt_ref, s_ref):
    acc = p_ref[0] + p_ref[1]
    numer = acc[:, 0:HC]
    den = acc[:, HC:WT1_COLS]
    dsafe = jnp.where(den == 0.0, 1.0, den)
    rid = lax.broadcasted_iota(jnp.int32, (LANES, HC), 0)
    cid = lax.broadcasted_iota(jnp.int32, (LANES, HC), 1)
    rep = jnp.where((cid // HID) == rid, 1.0, 0.0)
    denb = jnp.dot(dsafe, rep, precision=_HIGH)
    v = numer / denb + b1_ref[...]
    h1 = jnp.where(v > 0.0, v, jnp.exp(v) - 1.0)
    wh2 = jnp.dot(h1, w2_ref[...], precision=_HIGH)
    sf = jnp.dot(wh2, as_ref[...], precision=_HIGH)
    tf = jnp.dot(wh2, at_ref[...], precision=_HIGH)
    wt_ref[...] = jnp.concatenate([wh2, tf], axis=1)
    s_ref[...] = sf


def _tc2(part1, b1, w2r, as2, at2):
    bn = 1000
    grid = (N // bn,)
    return pl.pallas_call(
        _tc2_body,
        grid=grid,
        in_specs=[
            pl.BlockSpec((2, bn, ACC1_COLS), lambda i: (0, i, 0)),
            pl.BlockSpec((1, HC), lambda i: (0, 0)),
            pl.BlockSpec((HC, D_OUT), lambda i: (0, 0)),
            pl.BlockSpec((D_OUT, LANES), lambda i: (0, 0)),
            pl.BlockSpec((D_OUT, LANES), lambda i: (0, 0)),
        ],
        out_specs=[
            pl.BlockSpec((bn, WT2_COLS), lambda i: (i, 0)),
            pl.BlockSpec((bn, SCOLS), lambda i: (i, 0)),
        ],
        out_shape=[
            jax.ShapeDtypeStruct((N, WT2_COLS), jnp.float32),
            jax.ShapeDtypeStruct((N, SCOLS), jnp.float32),
        ],
    )(part1, b1, w2r, as2, at2)


def _zeros16():
    return jnp.zeros((LANES,), jnp.float32)


EPT = E // NTILES


def _sc_edge(ei, wt, s, wt_cols, acc_cols, heads, t_col, ebp):
    nj = EPT // ebp
    assert EPT % ebp == 0 and ebp % 8 == 0
    mesh = plsc.VectorSubcoreMesh(core_axis_name="c", subcore_axis_name="s")

    @functools.partial(
        pl.kernel,
        out_type=jax.ShapeDtypeStruct((2, ACC_ROWS, acc_cols), jnp.float32),
        mesh=mesh,
        scratch_types=[
            pltpu.VMEM((2, ebp), jnp.int32),
            pltpu.VMEM((2, ebp), jnp.int32),
            pltpu.VMEM((2, ebp), jnp.int32),
            pltpu.VMEM((2, ebp, SCOLS), jnp.float32),
            pltpu.VMEM((2, ebp, wt_cols), jnp.float32),
            pltpu.VMEM((2, ebp, acc_cols), jnp.float32),
            pltpu.VMEM_SHARED((ACC_ROWS, acc_cols), jnp.float32),
            pltpu.SemaphoreType.DMA,
            pltpu.SemaphoreType.DMA,
            pltpu.SemaphoreType.DMA,
            pltpu.SemaphoreType.DMA,
            pltpu.SemaphoreType.DMA,
            pltpu.SemaphoreType.DMA,
        ],
        compiler_params=_SC_PARAMS,
    )
    def edge_kernel(ei_hbm, wt_hbm, s_hbm, out_hbm,
                    dstb_v, srcb_v, dsts_v, sb_v, wt_v, msg_v, acc_sh,
                    isem0, isem1, gsem0, gsem1, ssem0, ssem1):
        cid = lax.axis_index("c")
        sid = lax.axis_index("s")
        wid = cid * 16 + sid
        isems = (isem0, isem1)
        gsems = (gsem0, gsem1)
        ssems = (ssem0, ssem1)
        ebase = wid * EPT

        zb = msg_v.at[0]

        @pl.loop(0, ebp)
        def _(r):
            for c in range(0, acc_cols, LANES):
                zb[r, pl.ds(c, LANES)] = _zeros16()

        base_row = sid * ROWS_PER_TILE
        full, rem = divmod(ROWS_PER_TILE, ebp)
        for i in range(full):
            pltpu.sync_copy(zb, acc_sh.at[pl.ds(base_row + i * ebp, ebp)])
        if rem:
            pltpu.sync_copy(zb.at[pl.ds(0, rem)],
                            acc_sh.at[pl.ds(base_row + full * ebp, rem)])
        plsc.subcore_barrier()

        def start_idx(slot, jb):
            base = pl.multiple_of(ebase + jb * ebp, 8)
            pltpu.async_copy(ei_hbm.at[0, pl.ds(base, ebp)],
                             dstb_v.at[slot], isems[slot])
            pltpu.async_copy(ei_hbm.at[1, pl.ds(base, ebp)],
                             srcb_v.at[slot], isems[slot])

        def wait_idx(slot, jb):
            base = pl.multiple_of(ebase + jb * ebp, 8)
            pltpu.make_async_copy(ei_hbm.at[0, pl.ds(base, ebp)],
                                  dstb_v.at[slot], isems[slot]).wait()
            pltpu.make_async_copy(ei_hbm.at[1, pl.ds(base, ebp)],
                                  srcb_v.at[slot], isems[slot]).wait()

        def fetch(slot):
            pltpu.async_copy(s_hbm.at[dstb_v.at[slot]], sb_v.at[slot],
                             gsems[slot])
            pltpu.async_copy(wt_hbm.at[srcb_v.at[slot]], wt_v.at[slot],
                             gsems[slot])

        def wait_fetch(slot):
            pltpu.make_async_copy(s_hbm.at[dstb_v.at[slot]], sb_v.at[slot],
                                  gsems[slot]).wait()
            pltpu.make_async_copy(wt_hbm.at[srcb_v.at[slot]], wt_v.at[slot],
                                  gsems[slot]).wait()

        _copy_offs = sorted(set(list(range(0, ebp - 15, 16)) + [ebp - 16]))

        def snap_idx(slot):
            for off in _copy_offs:
                dsts_v.at[slot][pl.ds(off, LANES)] = \
                    dstb_v.at[slot][pl.ds(off, LANES)]

        def compute(slot):
            sbs, wts, msgs = sb_v.at[slot], wt_v.at[slot], msg_v.at[slot]

            @plsc.parallel_loop(0, ebp, unroll=4)
            def _(k):
                sv = sbs[k, pl.ds(0, LANES)]
                tv = wts[k, pl.ds(t_col, LANES)]
                z = sv + tv
                w = jnp.exp(jnp.maximum(z, 0.2 * z))
                msgs[k, pl.ds(t_col, LANES)] = w
                for h in range(heads):
                    wspl = lax.gather(
                        w, jnp.full((LANES, 1), h, jnp.int32),
                        lax.GatherDimensionNumbers(
                            offset_dims=(), collapsed_slice_dims=(0,),
                            start_index_map=(0,)),
                        slice_sizes=(1,),
                        mode=lax.GatherScatterMode.PROMISE_IN_BOUNDS)
                    sl = pl.ds(h * LANES, LANES)
                    msgs[k, sl] = wts[k, sl] * wspl

        def start_scatter(slot):
            pltpu.async_copy(msg_v.at[slot], acc_sh.at[dsts_v.at[slot]],
                             ssems[slot], add=True)

        def wait_scatter(slot):
            pltpu.make_async_copy(msg_v.at[slot],
                                  acc_sh.at[dsts_v.at[slot]],
                                  ssems[slot]).wait()

        start_idx(0, jnp.int32(0))
        start_idx(1, jnp.int32(1))
        wait_idx(0, jnp.int32(0))
        fetch(0)
        wait_idx(1, jnp.int32(1))
        fetch(1)

        @pl.loop(0, (nj - 1) // 2)
        def _(t):
            j0 = 2 * t
            j1 = 2 * t + 1
            wait_fetch(0)

            @pl.when(t > 0)
            def _():
                wait_scatter(0)

            snap_idx(0)
            start_idx(0, j0 + 2)
            compute(0)
            start_scatter(0)
            wait_idx(0, j0 + 2)
            fetch(0)

            wait_fetch(1)

            @pl.when(t > 0)
            def _():
                wait_scatter(1)

            snap_idx(1)

            @pl.when(j1 + 2 < nj)
            def _():
                start_idx(1, j1 + 2)

            compute(1)
            start_scatter(1)

            @pl.when(j1 + 2 < nj)
            def _():
                wait_idx(1, j1 + 2)
                fetch(1)

        if nj % 2:
            wait_fetch(0)
            wait_scatter(0)
            snap_idx(0)
            compute(0)
            start_scatter(0)
            wait_scatter(1)
            wait_scatter(0)
        else:
            wait_fetch(0)
            wait_scatter(0)
            snap_idx(0)
            compute(0)
            start_scatter(0)
            wait_fetch(1)
            wait_scatter(1)
            snap_idx(1)
            compute(1)
            start_scatter(1)
            wait_scatter(0)
            wait_scatter(1)
        plsc.subcore_barrier()
        pltpu.sync_copy(acc_sh.at[pl.ds(base_row, ROWS_PER_TILE)],
                        out_hbm.at[cid, pl.ds(base_row, ROWS_PER_TILE)])

    return edge_kernel(ei, wt, s)


def _sc_final(p2a, p2b, index, b2):
    mesh = plsc.VectorSubcoreMesh(core_axis_name="c", subcore_axis_name="s")
    KB = 80
    nblk = N // KB

    @functools.partial(
        pl.kernel,
        out_type=jax.ShapeDtypeStruct((N, D_OUT), jnp.float32),
        mesh=mesh,
        scratch_types=[
            pltpu.VMEM((KB,), jnp.int32),
            pltpu.VMEM((KB, ACC2_COLS), jnp.float32),
            pltpu.VMEM((KB, ACC2_COLS), jnp.float32),
            pltpu.VMEM((KB, D_OUT), jnp.float32),
            pltpu.VMEM((KB, LANES), jnp.float32),
            pltpu.VMEM((LANES,), jnp.float32),
        ],
        compiler_params=_SC_PARAMS,
    )
    def final_kernel(pa_hbm, pb_hbm, idx_hbm, b2_hbm, out_hbm,
                     i_v, ra_v, rb_v, o_v, d_v, b2_v):
        cid = lax.axis_index("c")
        sid = lax.axis_index("s")
        wid = cid * 16 + sid
        pltpu.sync_copy(b2_hbm, b2_v)

        nrounds = (nblk + NTILES - 1) // NTILES

        @pl.loop(0, nrounds)
        def _(j):
            b = j * NTILES + wid

            @pl.when(b < nblk)
            def _():
                base = b * KB
                pltpu.sync_copy(idx_hbm.at[pl.ds(base, KB)], i_v)
                pltpu.sync_copy(pa_hbm.at[i_v], ra_v)
                pltpu.sync_copy(pb_hbm.at[i_v], rb_v)

                @pl.loop(0, KB)
                def _(k):
                    nv = ra_v[k, pl.ds(0, LANES)] + rb_v[k, pl.ds(0, LANES)]
                    dv = (ra_v[k, pl.ds(D_OUT, LANES)]
                          + rb_v[k, pl.ds(D_OUT, LANES)])
                    d_v[k, pl.ds(0, LANES)] = dv
                    i0 = jnp.full((LANES,), k, jnp.int32)
                    i1 = jnp.zeros((LANES,), jnp.int32)
                    dspl = plsc.load_gather(d_v, [i0, i1])
                    dsafe = jnp.where(dspl == 0.0, 1.0, dspl)
                    o_v[k, pl.ds(0, LANES)] = nv / dsafe + b2_v[pl.ds(0, LANES)]

                pltpu.sync_copy(o_v, out_hbm.at[pl.ds(base, KB)])

    return final_kernel(p2a, p2b, index, b2)


def kernel(x, edge_index, index, W1, a_s1, a_n1, b1, W2, a_s2, a_n2, b2):
    ei = edge_index.astype(jnp.int32)
    idx = index.astype(jnp.int32)
    w1r = jnp.transpose(W1, (1, 0, 2)).reshape(D_IN, HC)
    as1 = _head_select(a_s1)
    at1 = _head_select(a_n1)
    w2r = W2.reshape(HC, D_OUT)
    as2 = _head_select(a_s2)
    at2 = _head_select(a_n2)

    wt1, s1 = _tc1(x, w1r, as1, at1)
    part1 = _sc_edge(ei, wt1, s1, WT1_COLS, ACC1_COLS, HEADS, HC, 40)
    wt2, s2 = _tc2(part1[:, :N, :], b1.reshape(1, HC), w2r, as2, at2)
    part2 = _sc_edge(ei, wt2, s2, WT2_COLS, ACC2_COLS, 1, D_OUT, 80)
    return _sc_final(part2[0], part2[1], idx, b2)


# speedup vs baseline: 1.0009x; 1.0008x over previous
import dataclasses
import functools

import jax
import jax.numpy as jnp
from jax import lax
from jax.experimental import pallas as pl
from jax.experimental.pallas import tpu as pltpu
from jax.experimental.pallas import tpu_sc as plsc

N = 10000
E = 320000
D_IN = 128
HID = 16
HEADS = 8
D_OUT = 16
HC = HEADS * HID

LANES = 16
NTILES = 32
ACC_ROWS = 10112
ROWS_PER_TILE = ACC_ROWS // 16

WT1_COLS = 144
ACC1_COLS = 144
WT2_COLS = 32
ACC2_COLS = 32
SCOLS = 16

_HIGH = lax.Precision.HIGHEST

_SC_PARAMS = pltpu.CompilerParams(use_tc_tiling_on_sc=False)
if "needs_layout_passes" in pltpu.CompilerParams.__dataclass_fields__:
    _SC_PARAMS = dataclasses.replace(_SC_PARAMS, needs_layout_passes=False)


def _head_select(a):
    h, f = a.shape
    rows = jnp.arange(h * f) // f
    mask = (rows[:, None] == jnp.arange(LANES)[None, :]).astype(a.dtype)
    return mask * a.reshape(-1)[:, None]


def _tc1_body(x_ref, w_ref, as_ref, at_ref, wt_ref, s_ref):
    xb = x_ref[...]
    wh = jnp.dot(xb, w_ref[...], precision=_HIGH)
    sf = jnp.dot(wh, as_ref[...], precision=_HIGH)
    tf = jnp.dot(wh, at_ref[...], precision=_HIGH)
    wt_ref[...] = jnp.concatenate([wh, tf], axis=1)
    s_ref[...] = sf


def _tc1(x, w1r, as1, at1):
    bn = 1000
    grid = (N // bn,)
    return pl.pallas_call(
        _tc1_body,
        grid=grid,
        in_specs=[
            pl.BlockSpec((bn, D_IN), lambda i: (i, 0)),
            pl.BlockSpec((D_IN, HC), lambda i: (0, 0)),
            pl.BlockSpec((HC, LANES), lambda i: (0, 0)),
            pl.BlockSpec((HC, LANES), lambda i: (0, 0)),
        ],
        out_specs=[
            pl.BlockSpec((bn, WT1_COLS), lambda i: (i, 0)),
            pl.BlockSpec((bn, SCOLS), lambda i: (i, 0)),
        ],
        out_shape=[
            jax.ShapeDtypeStruct((N, WT1_COLS), jnp.float32),
            jax.ShapeDtypeStruct((N, SCOLS), jnp.float32),
        ],
    )(x, w1r, as1, at1)


def _tc2_body(p_ref, b1_ref, w2_ref, as_ref, at_ref, wt_ref, s_ref):
    acc = p_ref[0] + p_ref[1]
    numer = acc[:, 0:HC]
    den = acc[:, HC:WT1_COLS]
    dsafe = jnp.where(den == 0.0, 1.0, den)
    rid = lax.broadcasted_iota(jnp.int32, (LANES, HC), 0)
    cid = lax.broadcasted_iota(jnp.int32, (LANES, HC), 1)
    rep = jnp.where((cid // HID) == rid, 1.0, 0.0)
    denb = jnp.dot(dsafe, rep, precision=_HIGH)
    v = numer / denb + b1_ref[...]
    h1 = jnp.where(v > 0.0, v, jnp.exp(v) - 1.0)
    wh2 = jnp.dot(h1, w2_ref[...], precision=_HIGH)
    sf = jnp.dot(wh2, as_ref[...], precision=_HIGH)
    tf = jnp.dot(wh2, at_ref[...], precision=_HIGH)
    wt_ref[...] = jnp.concatenate([wh2, tf], axis=1)
    s_ref[...] = sf


def _tc2(part1, b1, w2r, as2, at2):
    bn = 1000
    grid = (N // bn,)
    return pl.pallas_call(
        _tc2_body,
        grid=grid,
        in_specs=[
            pl.BlockSpec((2, bn, ACC1_COLS), lambda i: (0, i, 0)),
            pl.BlockSpec((1, HC), lambda i: (0, 0)),
            pl.BlockSpec((HC, D_OUT), lambda i: (0, 0)),
            pl.BlockSpec((D_OUT, LANES), lambda i: (0, 0)),
            pl.BlockSpec((D_OUT, LANES), lambda i: (0, 0)),
        ],
        out_specs=[
            pl.BlockSpec((bn, WT2_COLS), lambda i: (i, 0)),
            pl.BlockSpec((bn, SCOLS), lambda i: (i, 0)),
        ],
        out_shape=[
            jax.ShapeDtypeStruct((N, WT2_COLS), jnp.float32),
            jax.ShapeDtypeStruct((N, SCOLS), jnp.float32),
        ],
    )(part1, b1, w2r, as2, at2)


def _zeros16():
    return jnp.zeros((LANES,), jnp.float32)


EPT = E // NTILES


def _sc_edge(ei, wt, s, wt_cols, acc_cols, heads, t_col, ebp):
    nj = EPT // ebp
    assert EPT % ebp == 0 and ebp % 8 == 0
    mesh = plsc.VectorSubcoreMesh(core_axis_name="c", subcore_axis_name="s")

    @functools.partial(
        pl.kernel,
        out_type=jax.ShapeDtypeStruct((2, ACC_ROWS, acc_cols), jnp.float32),
        mesh=mesh,
        scratch_types=[
            pltpu.VMEM((2, ebp), jnp.int32),
            pltpu.VMEM((2, ebp), jnp.int32),
            pltpu.VMEM((2, ebp), jnp.int32),
            pltpu.VMEM((2, ebp, SCOLS), jnp.float32),
            pltpu.VMEM((2, ebp, wt_cols), jnp.float32),
            pltpu.VMEM((2, ebp, acc_cols), jnp.float32),
            pltpu.VMEM_SHARED((ACC_ROWS, acc_cols), jnp.float32),
            pltpu.SemaphoreType.DMA,
            pltpu.SemaphoreType.DMA,
            pltpu.SemaphoreType.DMA,
            pltpu.SemaphoreType.DMA,
            pltpu.SemaphoreType.DMA,
            pltpu.SemaphoreType.DMA,
        ],
        compiler_params=_SC_PARAMS,
    )
    def edge_kernel(ei_hbm, wt_hbm, s_hbm, out_hbm,
                    dstb_v, srcb_v, dsts_v, sb_v, wt_v, msg_v, acc_sh,
                    isem0, isem1, gsem0, gsem1, ssem0, ssem1):
        cid = lax.axis_index("c")
        sid = lax.axis_index("s")
        wid = cid * 16 + sid
        isems = (isem0, isem1)
        gsems = (gsem0, gsem1)
        ssems = (ssem0, ssem1)
        ebase = wid * EPT

        zb = msg_v.at[0]

        @pl.loop(0, ebp)
        def _(r):
            for c in range(0, acc_cols, LANES):
                zb[r, pl.ds(c, LANES)] = _zeros16()

        base_row = sid * ROWS_PER_TILE
        full, rem = divmod(ROWS_PER_TILE, ebp)
        for i in range(full):
            pltpu.sync_copy(zb, acc_sh.at[pl.ds(base_row + i * ebp, ebp)])
        if rem:
            pltpu.sync_copy(zb.at[pl.ds(0, rem)],
                            acc_sh.at[pl.ds(base_row + full * ebp, rem)])
        plsc.subcore_barrier()

        def start_idx(slot, jb):
            base = pl.multiple_of(ebase + jb * ebp, 8)
            pltpu.async_copy(ei_hbm.at[0, pl.ds(base, ebp)],
                             dstb_v.at[slot], isems[slot])
            pltpu.async_copy(ei_hbm.at[1, pl.ds(base, ebp)],
                             srcb_v.at[slot], isems[slot])

        def wait_idx(slot, jb):
            base = pl.multiple_of(ebase + jb * ebp, 8)
            pltpu.make_async_copy(ei_hbm.at[0, pl.ds(base, ebp)],
                                  dstb_v.at[slot], isems[slot]).wait()
            pltpu.make_async_copy(ei_hbm.at[1, pl.ds(base, ebp)],
                                  srcb_v.at[slot], isems[slot]).wait()

        def fetch(slot):
            pltpu.async_copy(s_hbm.at[dstb_v.at[slot]], sb_v.at[slot],
                             gsems[slot])
            pltpu.async_copy(wt_hbm.at[srcb_v.at[slot]], wt_v.at[slot],
                             gsems[slot])

        def wait_fetch(slot):
            pltpu.make_async_copy(s_hbm.at[dstb_v.at[slot]], sb_v.at[slot],
                                  gsems[slot]).wait()
            pltpu.make_async_copy(wt_hbm.at[srcb_v.at[slot]], wt_v.at[slot],
                                  gsems[slot]).wait()

        _copy_offs = sorted(set(list(range(0, ebp - 15, 16)) + [ebp - 16]))

        def snap_idx(slot):
            for off in _copy_offs:
                dsts_v.at[slot][pl.ds(off, LANES)] = \
                    dstb_v.at[slot][pl.ds(off, LANES)]

        def compute(slot):
            sbs, wts, msgs = sb_v.at[slot], wt_v.at[slot], msg_v.at[slot]

            @plsc.parallel_loop(0, ebp, unroll=4)
            def _(k):
                sv = sbs[k, pl.ds(0, LANES)]
                tv = wts[k, pl.ds(t_col, LANES)]
                z = sv + tv
                w = jnp.exp(jnp.maximum(z, 0.2 * z))
                msgs[k, pl.ds(t_col, LANES)] = w
                for h in range(heads):
                    wspl = lax.gather(
                        w, jnp.full((LANES, 1), h, jnp.int32),
                        lax.GatherDimensionNumbers(
                            offset_dims=(), collapsed_slice_dims=(0,),
                            start_index_map=(0,)),
                        slice_sizes=(1,),
                        mode=lax.GatherScatterMode.PROMISE_IN_BOUNDS)
                    sl = pl.ds(h * LANES, LANES)
                    msgs[k, sl] = wts[k, sl] * wspl

        def start_scatter(slot):
            pltpu.async_copy(msg_v.at[slot], acc_sh.at[dsts_v.at[slot]],
                             ssems[slot], add=True)

        def wait_scatter(slot):
            pltpu.make_async_copy(msg_v.at[slot],
                                  acc_sh.at[dsts_v.at[slot]],
                                  ssems[slot]).wait()

        start_idx(0, jnp.int32(0))
        start_idx(1, jnp.int32(1))
        wait_idx(0, jnp.int32(0))
        fetch(0)
        wait_idx(1, jnp.int32(1))
        fetch(1)

        @pl.loop(0, (nj - 1) // 2)
        def _(t):
            j0 = 2 * t
            j1 = 2 * t + 1
            wait_fetch(0)

            @pl.when(t > 0)
            def _():
                wait_scatter(0)

            snap_idx(0)
            start_idx(0, j0 + 2)
            compute(0)
            start_scatter(0)
            wait_idx(0, j0 + 2)
            fetch(0)

            wait_fetch(1)

            @pl.when(t > 0)
            def _():
                wait_scatter(1)

            snap_idx(1)

            @pl.when(j1 + 2 < nj)
            def _():
                start_idx(1, j1 + 2)

            compute(1)
            start_scatter(1)

            @pl.when(j1 + 2 < nj)
            def _():
                wait_idx(1, j1 + 2)
                fetch(1)

        if nj % 2:
            wait_fetch(0)
            wait_scatter(0)
            snap_idx(0)
            compute(0)
            start_scatter(0)
            wait_scatter(1)
            wait_scatter(0)
        else:
            wait_fetch(0)
            wait_scatter(0)
            snap_idx(0)
            compute(0)
            start_scatter(0)
            wait_fetch(1)
            wait_scatter(1)
            snap_idx(1)
            compute(1)
            start_scatter(1)
            wait_scatter(0)
            wait_scatter(1)
        plsc.subcore_barrier()
        pltpu.sync_copy(acc_sh.at[pl.ds(base_row, ROWS_PER_TILE)],
                        out_hbm.at[cid, pl.ds(base_row, ROWS_PER_TILE)])

    return edge_kernel(ei, wt, s)


def _sc_final(p2a, p2b, index, b2):
    mesh = plsc.VectorSubcoreMesh(core_axis_name="c", subcore_axis_name="s")
    KB = 80
    nblk = N // KB

    @functools.partial(
        pl.kernel,
        out_type=jax.ShapeDtypeStruct((N, D_OUT), jnp.float32),
        mesh=mesh,
        scratch_types=[
            pltpu.VMEM((KB,), jnp.int32),
            pltpu.VMEM((KB, ACC2_COLS), jnp.float32),
            pltpu.VMEM((KB, ACC2_COLS), jnp.float32),
            pltpu.VMEM((KB, D_OUT), jnp.float32),
            pltpu.VMEM((KB, LANES), jnp.float32),
            pltpu.VMEM((LANES,), jnp.float32),
        ],
        compiler_params=_SC_PARAMS,
    )
    def final_kernel(pa_hbm, pb_hbm, idx_hbm, b2_hbm, out_hbm,
                     i_v, ra_v, rb_v, o_v, d_v, b2_v):
        cid = lax.axis_index("c")
        sid = lax.axis_index("s")
        wid = cid * 16 + sid
        pltpu.sync_copy(b2_hbm, b2_v)

        nrounds = (nblk + NTILES - 1) // NTILES

        @pl.loop(0, nrounds)
        def _(j):
            b = j * NTILES + wid

            @pl.when(b < nblk)
            def _():
                base = b * KB
                pltpu.sync_copy(idx_hbm.at[pl.ds(base, KB)], i_v)
                pltpu.sync_copy(pa_hbm.at[i_v], ra_v)
                pltpu.sync_copy(pb_hbm.at[i_v], rb_v)

                @pl.loop(0, KB)
                def _(k):
                    nv = ra_v[k, pl.ds(0, LANES)] + rb_v[k, pl.ds(0, LANES)]
                    dv = (ra_v[k, pl.ds(D_OUT, LANES)]
                          + rb_v[k, pl.ds(D_OUT, LANES)])
                    d_v[k, pl.ds(0, LANES)] = dv
                    i0 = jnp.full((LANES,), k, jnp.int32)
                    i1 = jnp.zeros((LANES,), jnp.int32)
                    dspl = plsc.load_gather(d_v, [i0, i1])
                    dsafe = jnp.where(dspl == 0.0, 1.0, dspl)
                    o_v[k, pl.ds(0, LANES)] = nv / dsafe + b2_v[pl.ds(0, LANES)]

                pltpu.sync_copy(o_v, out_hbm.at[pl.ds(base, KB)])

    return final_kernel(p2a, p2b, index, b2)


def kernel(x, edge_index, index, W1, a_s1, a_n1, b1, W2, a_s2, a_n2, b2):
    ei = edge_index.astype(jnp.int32)
    idx = index.astype(jnp.int32)
    w1r = jnp.transpose(W1, (1, 0, 2)).reshape(D_IN, HC)
    as1 = _head_select(a_s1)
    at1 = _head_select(a_n1)
    w2r = W2.reshape(HC, D_OUT)
    as2 = _head_select(a_s2)
    at2 = _head_select(a_n2)

    wt1, s1 = _tc1(x, w1r, as1, at1)
    part1 = _sc_edge(ei, wt1, s1, WT1_COLS, ACC1_COLS, HEADS, HC, 40)
    wt2, s2 = _tc2(part1[:, :N, :], b1.reshape(1, HC), w2r, as2, at2)
    part2 = _sc_edge(ei, wt2, s2, WT2_COLS, ACC2_COLS, 1, D_OUT, 80)
    return _sc_final(part2[0], part2[1], idx, b2)
